# Initial kernel scaffold; baseline (speedup 1.0000x reference)
import jax
import jax.numpy as jnp
from jax.experimental import pallas as pl


def kernel(card_ids, card_stats, emb_table, W_stat, b_stat, W_comb, b_comb):
    raise NotImplementedError("write your pallas kernel here")


# speedup vs baseline: 1.5973x; 1.5973x over previous
import functools

import jax
import jax.numpy as jnp
from jax import lax
from jax.experimental import pallas as pl
from jax.experimental.pallas import tpu as pltpu
from jax.experimental.pallas import tpu_sc as plsc

D_HALF = 64
D_MODEL = 128
N_TOKENS = 4096 * 50

NW = 32
BPW = N_TOKENS // NW
IDXW = 128
IDX_ROWS = BPW // IDXW
CHUNK = 640
IDX_PER_CHUNK = CHUNK // IDXW
NCHUNK = BPW // CHUNK


def _gather_sc(table, ids_flat):
    mesh = plsc.VectorSubcoreMesh(core_axis_name="c", subcore_axis_name="s")

    @functools.partial(
        pl.kernel,
        mesh=mesh,
        compiler_params=pltpu.CompilerParams(use_tc_tiling_on_sc=False),
        out_type=jax.ShapeDtypeStruct((N_TOKENS, D_HALF), jnp.float32),
        scratch_types=[
            pltpu.VMEM((BPW,), jnp.int32),
            pltpu.VMEM((CHUNK, D_HALF), jnp.float32),
            pltpu.VMEM((CHUNK, D_HALF), jnp.float32),
            pltpu.SemaphoreType.DMA,
            pltpu.SemaphoreType.DMA,
            pltpu.SemaphoreType.DMA,
        ],
    )
    def k(table_hbm, ids_hbm, out_hbm, idx_v, buf0, buf1, gsem, osem0, osem1):
        wid = lax.axis_index("s") * 2 + lax.axis_index("c")
        base = wid * BPW
        pltpu.sync_copy(ids_hbm.at[pl.ds(base, BPW)], idx_v)
        bufs = (buf0, buf1)
        osems = (osem0, osem1)

        def issue(c, buf):
            hs = []
            for j in range(IDX_PER_CHUNK):
                hs.append(pltpu.async_copy(
                    table_hbm.at[idx_v.at[pl.ds((c * IDX_PER_CHUNK + j) * IDXW, IDXW)]],
                    buf.at[pl.ds(j * IDXW, IDXW)],
                    gsem))
            return hs

        pending = issue(0, bufs[0])
        out_h = [None, None]
        for c in range(NCHUNK):
            b = c & 1
            for h in pending:
                h.wait()
            if c + 1 < NCHUNK:
                if out_h[1 - b] is not None:
                    out_h[1 - b].wait()
                pending = issue(c + 1, bufs[1 - b])
            out_h[b] = pltpu.async_copy(
                bufs[b], out_hbm.at[pl.ds(base + c * CHUNK, CHUNK)], osems[b])
        out_h[(NCHUNK - 1) & 1].wait()

    return k(table, ids_flat)


def _dense_tc(id_emb, stats2d, W_stat, b_stat, W_comb, b_comb):
    ROWS = 2048
    grid = (N_TOKENS // ROWS,)

    def body(id_ref, st_ref, ws_ref, bs_ref, wc_ref, bc_ref, out_ref):
        s = jnp.dot(st_ref[...], ws_ref[...],
                    preferred_element_type=jnp.float32) + bs_ref[...]
        s = 0.5 * s * (1.0 + lax.erf(s * 0.7071067811865476))
        top = jnp.dot(id_ref[...], wc_ref[:D_HALF, :],
                      preferred_element_type=jnp.float32)
        bot = jnp.dot(s, wc_ref[D_HALF:, :],
                      preferred_element_type=jnp.float32)
        out_ref[...] = top + bot + bc_ref[...]

    return pl.pallas_call(
        body,
        grid=grid,
        in_specs=[
            pl.BlockSpec((ROWS, D_HALF), lambda i: (i, 0)),
            pl.BlockSpec((ROWS, 10), lambda i: (i, 0)),
            pl.BlockSpec((10, D_HALF), lambda i: (0, 0)),
            pl.BlockSpec((1, D_HALF), lambda i: (0, 0)),
            pl.BlockSpec((D_MODEL, D_MODEL), lambda i: (0, 0)),
            pl.BlockSpec((1, D_MODEL), lambda i: (0, 0)),
        ],
        out_specs=pl.BlockSpec((ROWS, D_MODEL), lambda i: (i, 0)),
        out_shape=jax.ShapeDtypeStruct((N_TOKENS, D_MODEL), jnp.float32),
    )(id_emb, stats2d, W_stat, b_stat.reshape(1, D_HALF),
      W_comb, b_comb.reshape(1, D_MODEL))


def kernel(card_ids, card_stats, emb_table, W_stat, b_stat, W_comb, b_comb):
    B, L = card_ids.shape
    ids_flat = card_ids.reshape(N_TOKENS).astype(jnp.int32)
    id_emb = _gather_sc(emb_table, ids_flat)
    stats2d = card_stats.reshape(N_TOKENS, 10)
    out = _dense_tc(id_emb, stats2d, W_stat, b_stat, W_comb, b_comb)
    return out.reshape(B, L, D_MODEL)


# speedup vs baseline: 1.7802x; 1.1145x over previous
import functools

import jax
import jax.numpy as jnp
from jax import lax
from jax.experimental import pallas as pl
from jax.experimental.pallas import tpu as pltpu
from jax.experimental.pallas import tpu_sc as plsc

VOCAB = 100000
D_HALF = 64
D_MODEL = 128
N_TOKENS = 4096 * 50

NW = 32
BPW = N_TOKENS // NW
IDXW = 128
CHUNK = 256
IDX_PER_CHUNK = CHUNK // IDXW
NCHUNK = BPW // CHUNK


def _table_transform_tc(emb_table, W_comb):
    BR = 4000
    grid = (VOCAB // BR,)

    def body(t_ref, w_ref, o_ref):
        o_ref[...] = jnp.dot(t_ref[...], w_ref[:D_HALF, :],
                             preferred_element_type=jnp.float32)

    return pl.pallas_call(
        body,
        grid=grid,
        in_specs=[
            pl.BlockSpec((BR, D_HALF), lambda i: (i, 0)),
            pl.BlockSpec((D_MODEL, D_MODEL), lambda i: (0, 0)),
        ],
        out_specs=pl.BlockSpec((BR, D_MODEL), lambda i: (i, 0)),
        out_shape=jax.ShapeDtypeStruct((VOCAB, D_MODEL), jnp.float32),
    )(emb_table, W_comb)


def _gather_sc(table_t, ids_flat):
    mesh = plsc.VectorSubcoreMesh(core_axis_name="c", subcore_axis_name="s")

    @functools.partial(
        pl.kernel,
        mesh=mesh,
        out_type=jax.ShapeDtypeStruct((N_TOKENS, D_MODEL), jnp.float32),
        scratch_types=[
            pltpu.VMEM((BPW,), jnp.int32),
            pltpu.VMEM((CHUNK, D_MODEL), jnp.float32),
            pltpu.VMEM((CHUNK, D_MODEL), jnp.float32),
            pltpu.SemaphoreType.DMA,
            pltpu.SemaphoreType.DMA,
            pltpu.SemaphoreType.DMA,
        ],
    )
    def k(table_hbm, ids_hbm, out_hbm, idx_v, buf0, buf1, gsem, osem0, osem1):
        wid = lax.axis_index("s") * 2 + lax.axis_index("c")
        base = wid * BPW
        pltpu.sync_copy(ids_hbm.at[pl.ds(base, BPW)], idx_v)
        bufs = (buf0, buf1)
        osems = (osem0, osem1)

        def issue(c, buf):
            hs = []
            for j in range(IDX_PER_CHUNK):
                hs.append(pltpu.async_copy(
                    table_hbm.at[idx_v.at[pl.ds((c * IDX_PER_CHUNK + j) * IDXW, IDXW)]],
                    buf.at[pl.ds(j * IDXW, IDXW)],
                    gsem))
            return hs

        pending = issue(0, bufs[0])
        out_h = [None, None]
        for c in range(NCHUNK):
            b = c & 1
            for h in pending:
                h.wait()
            if c + 1 < NCHUNK:
                if out_h[1 - b] is not None:
                    out_h[1 - b].wait()
                pending = issue(c + 1, bufs[1 - b])
            out_h[b] = pltpu.async_copy(
                bufs[b], out_hbm.at[pl.ds(base + c * CHUNK, CHUNK)], osems[b])
        out_h[(NCHUNK - 1) & 1].wait()

    return k(table_t, ids_flat)


def _dense_tc(gathered, stats2d, W_stat, b_stat, W_comb, b_comb):
    ROWS = 2048
    grid = (N_TOKENS // ROWS,)

    def body(g_ref, st_ref, ws_ref, bs_ref, wc_ref, bc_ref, out_ref):
        s = jnp.dot(st_ref[...], ws_ref[...],
                    preferred_element_type=jnp.float32) + bs_ref[...]
        s = 0.5 * s * (1.0 + lax.erf(s * 0.7071067811865476))
        bot = jnp.dot(s, wc_ref[D_HALF:, :],
                      preferred_element_type=jnp.float32)
        out_ref[...] = g_ref[...] + bot + bc_ref[...]

    return pl.pallas_call(
        body,
        grid=grid,
        in_specs=[
            pl.BlockSpec((ROWS, D_MODEL), lambda i: (i, 0)),
            pl.BlockSpec((ROWS, 10), lambda i: (i, 0)),
            pl.BlockSpec((10, D_HALF), lambda i: (0, 0)),
            pl.BlockSpec((1, D_HALF), lambda i: (0, 0)),
            pl.BlockSpec((D_MODEL, D_MODEL), lambda i: (0, 0)),
            pl.BlockSpec((1, D_MODEL), lambda i: (0, 0)),
        ],
        out_specs=pl.BlockSpec((ROWS, D_MODEL), lambda i: (i, 0)),
        out_shape=jax.ShapeDtypeStruct((N_TOKENS, D_MODEL), jnp.float32),
    )(gathered, stats2d, W_stat, b_stat.reshape(1, D_HALF),
      W_comb, b_comb.reshape(1, D_MODEL))


def kernel(card_ids, card_stats, emb_table, W_stat, b_stat, W_comb, b_comb):
    B, L = card_ids.shape
    ids_flat = card_ids.reshape(N_TOKENS).astype(jnp.int32)
    table_t = _table_transform_tc(emb_table, W_comb)
    gathered = _gather_sc(table_t, ids_flat)
    stats2d = card_stats.reshape(N_TOKENS, 10)
    out = _dense_tc(gathered, stats2d, W_stat, b_stat, W_comb, b_comb)
    return out.reshape(B, L, D_MODEL)
